# baseline (speedup 1.0000x reference)
import functools

import jax
import jax.numpy as jnp
from jax import lax
from jax.experimental import pallas as pl
from jax.experimental.pallas import tpu as pltpu
from jax.experimental.pallas import tpu_sc as plsc

_NC = 2
_NS = 16
_NW = _NC * _NS
_KC = 128


def _transpose_body(e_ref, out_ref):
    out_ref[...] = e_ref[...].T


def _transpose(E, vb):
    D, V = E.shape
    return pl.pallas_call(
        _transpose_body,
        grid=(pl.cdiv(V, vb),),
        in_specs=[pl.BlockSpec((D, vb), lambda i: (0, i))],
        out_specs=pl.BlockSpec((vb, D), lambda i: (i, 0)),
        out_shape=jax.ShapeDtypeStruct((V, D), E.dtype),
    )(E)


def _make_pool(B, L, V, D):
    rpw = B * L // _NW
    ndw = B // _NW
    nch = rpw // _KC
    mesh = plsc.VectorSubcoreMesh(core_axis_name="c", subcore_axis_name="s")

    @functools.partial(
        pl.kernel,
        out_type=jax.ShapeDtypeStruct((B, D), jnp.float32),
        mesh=mesh,
        scratch_types=(
            [pltpu.VMEM((nch, _KC), jnp.int32)] * 2
            + [pltpu.VMEM((_KC, D), jnp.float32)] * 8
            + [pltpu.VMEM_SHARED((_NS * ndw, D), jnp.float32)]
            + [pltpu.SemaphoreType.DMA] * 16
        ),
        compiler_params=pltpu.CompilerParams(use_tc_tiling_on_sc=False),
    )
    def pool(et, docs3, dst3, zero2, out, si, di,
             r0, r1, r2, r3, r4, r5, r6, r7, acc,
             s0, s1, s2, s3, s4, s5, s6, s7,
             c0, c1, c2, c3, c4, c5, c6, c7):
        rs = [r0, r1, r2, r3, r4, r5, r6, r7]
        ss = [s0, s1, s2, s3, s4, s5, s6, s7]
        cs = [c0, c1, c2, c3, c4, c5, c6, c7]
        sid = lax.axis_index("s")
        wid = sid * _NC + lax.axis_index("c")
        pltpu.sync_copy(zero2, acc.at[pl.ds(sid * ndw, ndw)])
        pltpu.sync_copy(docs3.at[wid], si)
        pltpu.sync_copy(dst3.at[sid], di)
        for j in range(4):
            pltpu.async_copy(et.at[si.at[j]], rs[j], ss[j])

        @pl.loop(0, nch, step=8)
        def _(k):
            for j in range(8):
                kk = k + j
                j4 = (j + 4) % 8
                pltpu.make_async_copy(et.at[si.at[kk]], rs[j], ss[j]).wait()
                pltpu.async_copy(rs[j], acc.at[di.at[kk]], cs[j], add=True)

                @pl.when(kk + 4 < nch)
                def _():
                    @pl.when(kk >= 4)
                    def _():
                        pltpu.make_async_copy(
                            rs[j4], acc.at[di.at[kk]], cs[j4]).wait()

                    pltpu.async_copy(et.at[si.at[kk + 4]], rs[j4], ss[j4])

        for j in range(4, 8):
            pltpu.make_async_copy(rs[j], acc.at[di.at[0]], cs[j]).wait()
        pltpu.sync_copy(acc.at[pl.ds(sid * ndw, ndw)],
                        out.at[pl.ds(wid * ndw, ndw)])

    return pool, rpw, nch


def _mlp_body(x_ref, dl_ref, w1_ref, b1_ref, w2_ref, b2_ref, w3_ref, b3_ref,
              o_ref):
    x = x_ref[...] / dl_ref[...]
    h = jnp.maximum(jnp.dot(x, w1_ref[...]) + b1_ref[...], 0.0)
    h = jnp.maximum(jnp.dot(h, w2_ref[...]) + b2_ref[...], 0.0)
    o_ref[...] = jnp.dot(h, w3_ref[...]) + b3_ref[...]


def _mlp(x, dl, W1, b1, W2, b2, W3, b3, bb):
    B, D = x.shape
    H = W1.shape[1]
    C = W3.shape[1]
    full = lambda s: pl.BlockSpec(s, lambda i: (0, 0))
    return pl.pallas_call(
        _mlp_body,
        grid=(B // bb,),
        in_specs=[
            pl.BlockSpec((bb, D), lambda i: (i, 0)),
            pl.BlockSpec((bb, 1), lambda i: (i, 0)),
            full((D, H)), full((1, H)),
            full((H, H)), full((1, H)),
            full((H, C)), full((1, C)),
        ],
        out_specs=pl.BlockSpec((bb, C), lambda i: (i, 0)),
        out_shape=jax.ShapeDtypeStruct((B, C), jnp.float32),
    )(x, dl, W1, b1.reshape(1, H), W2, b2.reshape(1, H), W3, b3.reshape(1, C))


def kernel(docs, embeddings_matrix, doc_lens, W1, b1, W2, b2, W3, b3):
    B, L = docs.shape
    D, V = embeddings_matrix.shape

    ET = _transpose(embeddings_matrix, vb=2048)

    pool, rpw, nch = _make_pool(B, L, V, D)
    docs3 = docs.reshape(_NW, nch, _KC)
    ndw = B // _NW
    local = (jnp.arange(rpw, dtype=jnp.int32) // L).reshape(1, nch, _KC)
    dst3 = local + (jnp.arange(_NS, dtype=jnp.int32) * ndw).reshape(_NS, 1, 1)
    zero2 = jnp.zeros((ndw, D), jnp.float32)
    sums = pool(ET, docs3, dst3, zero2)

    return _mlp(sums, doc_lens.reshape(B, 1), W1, b1, W2, b2, W3, b3, bb=1024)


# speedup vs baseline: 1.4175x; 1.4175x over previous
import functools

import jax
import jax.numpy as jnp
from jax import lax
from jax.experimental import pallas as pl
from jax.experimental.pallas import tpu as pltpu
from jax.experimental.pallas import tpu_sc as plsc

_NC = 2
_NS = 16
_NW = _NC * _NS
_KC = 100

_PERM = (list(range(0, 32, 2)) + list(range(1, 32, 2))
         + list(range(32, 64, 2)) + list(range(33, 64, 2)))


def _transpose_body(e_ref, out_ref):
    out_ref[...] = e_ref[...].T.astype(jnp.bfloat16)


def _transpose(E, vb):
    D, V = E.shape
    return pl.pallas_call(
        _transpose_body,
        grid=(pl.cdiv(V, vb),),
        in_specs=[pl.BlockSpec((D, vb), lambda i: (0, i))],
        out_specs=pl.BlockSpec((vb, D), lambda i: (i, 0)),
        out_shape=jax.ShapeDtypeStruct((V, D), jnp.bfloat16),
    )(E)


def _make_pool(B, L, V, D):
    rpw = B * L // _NW
    ndw = B // _NW
    nch = rpw // _KC
    nbuf = 8
    mesh = plsc.VectorSubcoreMesh(core_axis_name="c", subcore_axis_name="s")

    @functools.partial(
        pl.kernel,
        out_type=jax.ShapeDtypeStruct((B, D), jnp.float32),
        mesh=mesh,
        scratch_types=(
            [pltpu.VMEM((nch, _KC), jnp.int32)]
            + [pltpu.VMEM((_KC, D), jnp.bfloat16)] * nbuf
            + [pltpu.VMEM((ndw, D), jnp.float32)]
            + [pltpu.SemaphoreType.DMA] * nbuf
        ),
        compiler_params=pltpu.CompilerParams(use_tc_tiling_on_sc=False,
                                             needs_layout_passes=False),
    )
    def pool(et, docs3, out, si,
             r0, r1, r2, r3, r4, r5, r6, r7, acc,
             s0, s1, s2, s3, s4, s5, s6, s7):
        rs = [r0, r1, r2, r3, r4, r5, r6, r7]
        ss = [s0, s1, s2, s3, s4, s5, s6, s7]
        sid = lax.axis_index("s")
        wid = sid * _NC + lax.axis_index("c")
        pltpu.sync_copy(docs3.at[wid], si)
        for j in range(nbuf - 1):
            pltpu.async_copy(et.at[si.at[j]], rs[j], ss[j])

        @pl.loop(0, nch, step=nbuf)
        def _(k):
            for j in range(nbuf):
                kk = k + j
                jn = (j + nbuf - 1) % nbuf
                pltpu.make_async_copy(et.at[si.at[kk]], rs[j], ss[j]).wait()
                rbuf = rs[j]

                def row_sum(r, c):
                    a0, a1, a2, a3 = c
                    hi_mask = jnp.full((16,), -65536, jnp.int32)
                    w0 = plsc.bitcast(rbuf[r, pl.ds(0, 32)], jnp.int32)
                    w1 = plsc.bitcast(rbuf[r, pl.ds(32, 32)], jnp.int32)
                    a0 += plsc.bitcast(w0 << 16, jnp.float32)
                    a1 += plsc.bitcast(w0 & hi_mask, jnp.float32)
                    a2 += plsc.bitcast(w1 << 16, jnp.float32)
                    a3 += plsc.bitcast(w1 & hi_mask, jnp.float32)
                    return a0, a1, a2, a3

                z = jnp.zeros((16,), jnp.float32)
                a0, a1, a2, a3 = lax.fori_loop(0, _KC, row_sum, (z, z, z, z),
                                               unroll=4)
                doc = kk // 2

                @pl.when(kk % 2 == 0)
                def _():
                    acc[doc, pl.ds(0, 16)] = a0
                    acc[doc, pl.ds(16, 16)] = a1
                    acc[doc, pl.ds(32, 16)] = a2
                    acc[doc, pl.ds(48, 16)] = a3

                @pl.when(kk % 2 == 1)
                def _():
                    plsc.addupdate(acc.at[doc, pl.ds(0, 16)], a0)
                    plsc.addupdate(acc.at[doc, pl.ds(16, 16)], a1)
                    plsc.addupdate(acc.at[doc, pl.ds(32, 16)], a2)
                    plsc.addupdate(acc.at[doc, pl.ds(48, 16)], a3)

                @pl.when(kk + nbuf - 1 < nch)
                def _():
                    pltpu.async_copy(et.at[si.at[kk + nbuf - 1]],
                                     rs[jn], ss[jn])

        pltpu.sync_copy(acc, out.at[pl.ds(wid * ndw, ndw)])

    return pool, rpw, nch


def _mlp_body(x_ref, dl_ref, w1_ref, b1_ref, w2_ref, b2_ref, w3_ref, b3_ref,
              o_ref):
    x = x_ref[...] / dl_ref[...]
    h = jnp.maximum(jnp.dot(x, w1_ref[...]) + b1_ref[...], 0.0)
    h = jnp.maximum(jnp.dot(h, w2_ref[...]) + b2_ref[...], 0.0)
    o_ref[...] = jnp.dot(h, w3_ref[...]) + b3_ref[...]


def _mlp(x, dl, W1, b1, W2, b2, W3, b3, bb):
    B, D = x.shape
    H = W1.shape[1]
    C = W3.shape[1]
    full = lambda s: pl.BlockSpec(s, lambda i: (0, 0))
    return pl.pallas_call(
        _mlp_body,
        grid=(B // bb,),
        in_specs=[
            pl.BlockSpec((bb, D), lambda i: (i, 0)),
            pl.BlockSpec((bb, 1), lambda i: (i, 0)),
            full((D, H)), full((1, H)),
            full((H, H)), full((1, H)),
            full((H, C)), full((1, C)),
        ],
        out_specs=pl.BlockSpec((bb, C), lambda i: (i, 0)),
        out_shape=jax.ShapeDtypeStruct((B, C), jnp.float32),
    )(x, dl, W1, b1.reshape(1, H), W2, b2.reshape(1, H), W3, b3.reshape(1, C))


def kernel(docs, embeddings_matrix, doc_lens, W1, b1, W2, b2, W3, b3):
    B, L = docs.shape
    D, V = embeddings_matrix.shape

    ET = _transpose(embeddings_matrix, vb=2048)

    pool, rpw, nch = _make_pool(B, L, V, D)
    docs3 = docs.reshape(_NW, nch, _KC)
    sums = pool(ET, docs3)

    W1p = W1[jnp.array(_PERM), :]
    return _mlp(sums, doc_lens.reshape(B, 1), W1p, b1, W2, b2, W3, b3, bb=1024)


# speedup vs baseline: 1.5069x; 1.0631x over previous
import functools

import jax
import jax.numpy as jnp
from jax import lax
from jax.experimental import pallas as pl
from jax.experimental.pallas import tpu as pltpu
from jax.experimental.pallas import tpu_sc as plsc

_NC = 2
_NS = 16
_NW = _NC * _NS
_KC = 100

_PERM = (list(range(0, 32, 2)) + list(range(1, 32, 2))
         + list(range(32, 64, 2)) + list(range(33, 64, 2)))


def _transpose_body(e_ref, out_ref):
    out_ref[...] = e_ref[...].T.astype(jnp.bfloat16)


def _transpose(E, vb):
    D, V = E.shape
    return pl.pallas_call(
        _transpose_body,
        grid=(pl.cdiv(V, vb),),
        in_specs=[pl.BlockSpec((D, vb), lambda i: (0, i))],
        out_specs=pl.BlockSpec((vb, D), lambda i: (i, 0)),
        out_shape=jax.ShapeDtypeStruct((V, D), jnp.bfloat16),
    )(E)


def _make_pool(B, L, V, D):
    rpw = B * L // _NW
    ndw = B // _NW
    nch = rpw // _KC
    nbuf = 8
    mesh = plsc.VectorSubcoreMesh(core_axis_name="c", subcore_axis_name="s")

    @functools.partial(
        pl.kernel,
        out_type=jax.ShapeDtypeStruct((B, D), jnp.float32),
        mesh=mesh,
        scratch_types=(
            [pltpu.VMEM((nch, _KC), jnp.int32)]
            + [pltpu.VMEM((_KC, D), jnp.bfloat16)] * nbuf
            + [pltpu.VMEM((ndw, D), jnp.float32)]
            + [pltpu.SemaphoreType.DMA] * nbuf
        ),
        compiler_params=pltpu.CompilerParams(use_tc_tiling_on_sc=False,
                                             needs_layout_passes=False),
    )
    def pool(et, docs3, out, si,
             r0, r1, r2, r3, r4, r5, r6, r7, acc,
             s0, s1, s2, s3, s4, s5, s6, s7):
        rs = [r0, r1, r2, r3, r4, r5, r6, r7]
        ss = [s0, s1, s2, s3, s4, s5, s6, s7]
        sid = lax.axis_index("s")
        wid = sid * _NC + lax.axis_index("c")
        pltpu.sync_copy(docs3.at[wid], si)
        for j in range(nbuf - 1):
            pltpu.async_copy(et.at[si.at[j]], rs[j], ss[j])

        @pl.loop(0, nch, step=nbuf)
        def _(k):
            for j in range(nbuf):
                kk = k + j
                jn = (j + nbuf - 1) % nbuf
                pltpu.make_async_copy(et.at[si.at[kk]], rs[j], ss[j]).wait()
                rbuf = rs[j]

                def row_sum(r, c):
                    a0, a1, a2, a3 = c
                    hi_mask = jnp.full((16,), -65536, jnp.int32)
                    w0 = plsc.bitcast(rbuf[r, pl.ds(0, 32)], jnp.int32)
                    w1 = plsc.bitcast(rbuf[r, pl.ds(32, 32)], jnp.int32)
                    a0 += plsc.bitcast(w0 << 16, jnp.float32)
                    a1 += plsc.bitcast(w0 & hi_mask, jnp.float32)
                    a2 += plsc.bitcast(w1 << 16, jnp.float32)
                    a3 += plsc.bitcast(w1 & hi_mask, jnp.float32)
                    return a0, a1, a2, a3

                z = jnp.zeros((16,), jnp.float32)
                a0, a1, a2, a3 = lax.fori_loop(0, _KC, row_sum, (z, z, z, z),
                                               unroll=4)
                doc = kk // 2

                @pl.when(kk % 2 == 0)
                def _():
                    acc[doc, pl.ds(0, 16)] = a0
                    acc[doc, pl.ds(16, 16)] = a1
                    acc[doc, pl.ds(32, 16)] = a2
                    acc[doc, pl.ds(48, 16)] = a3

                @pl.when(kk % 2 == 1)
                def _():
                    plsc.addupdate(acc.at[doc, pl.ds(0, 16)], a0)
                    plsc.addupdate(acc.at[doc, pl.ds(16, 16)], a1)
                    plsc.addupdate(acc.at[doc, pl.ds(32, 16)], a2)
                    plsc.addupdate(acc.at[doc, pl.ds(48, 16)], a3)

                @pl.when(kk + nbuf - 1 < nch)
                def _():
                    pltpu.async_copy(et.at[si.at[kk + nbuf - 1]],
                                     rs[jn], ss[jn])

        pltpu.sync_copy(acc, out.at[pl.ds(wid * ndw, ndw)])

    return pool, rpw, nch


def _mlp_body(x_ref, dl_ref, w1_ref, b1_ref, w2_ref, b2_ref, w3_ref, b3_ref,
              o_ref):
    x = x_ref[...] / dl_ref[...]
    h = jnp.maximum(jnp.dot(x, w1_ref[...]) + b1_ref[...], 0.0)
    h = jnp.maximum(jnp.dot(h, w2_ref[...]) + b2_ref[...], 0.0)
    o_ref[...] = jnp.dot(h, w3_ref[...]) + b3_ref[...]


def _mlp(x, dl, W1, b1, W2, b2, W3, b3, bb):
    B, D = x.shape
    H = W1.shape[1]
    C = W3.shape[1]
    full = lambda s: pl.BlockSpec(s, lambda i: (0, 0))
    return pl.pallas_call(
        _mlp_body,
        grid=(B // bb,),
        in_specs=[
            pl.BlockSpec((bb, D), lambda i: (i, 0)),
            pl.BlockSpec((bb, 1), lambda i: (i, 0)),
            full((D, H)), full((1, H)),
            full((H, H)), full((1, H)),
            full((H, C)), full((1, C)),
        ],
        out_specs=pl.BlockSpec((bb, C), lambda i: (i, 0)),
        out_shape=jax.ShapeDtypeStruct((B, C), jnp.float32),
    )(x, dl, W1, b1.reshape(1, H), W2, b2.reshape(1, H), W3, b3.reshape(1, C))


def kernel(docs, embeddings_matrix, doc_lens, W1, b1, W2, b2, W3, b3):
    B, L = docs.shape
    D, V = embeddings_matrix.shape

    ET = embeddings_matrix.T.astype(jnp.bfloat16)

    pool, rpw, nch = _make_pool(B, L, V, D)
    docs3 = docs.reshape(_NW, nch, _KC)
    sums = pool(ET, docs3)

    W1p = W1[jnp.array(_PERM), :]
    return _mlp(sums, doc_lens.reshape(B, 1), W1p, b1, W2, b2, W3, b3, bb=1024)
